# initial kernel scaffold (unmeasured)
import jax
import jax.numpy as jnp
from jax import lax
from jax.experimental import pallas as pl
from jax.experimental.pallas import tpu as pltpu

N_DEV = 4
M, N = 4096, 8192
ROWS = 512
CHUNKS = M // ROWS
ROUNDS = CHUNKS // N_DEV


def _silu(y):
    return y * (1.0 / (1.0 + jnp.exp(-y)))


def kernel(x, w_mat):
    p = jnp.dot(x, w_mat, preferred_element_type=jnp.float32)

    def body(p_ref, o_ref, comm, local, send_sems, recv_sems, copy_sem,
             store_sem, credit_sem):
        my = lax.axis_index("i")
        left = jnp.mod(my - 1, N_DEV)
        right = jnp.mod(my + 1, N_DEV)

        barrier = pltpu.get_barrier_semaphore()
        for nbr in (left, right):
            pl.semaphore_signal(barrier, inc=1, device_id=(nbr,),
                                device_id_type=pl.DeviceIdType.MESH)
        pl.semaphore_wait(barrier, 2)

        step = 0
        for g in range(ROUNDS):
            ld = pltpu.make_async_copy(
                p_ref.at[pl.ds((g * N_DEV + my) * ROWS, ROWS)],
                comm.at[0], copy_sem)
            ld.start()
            ld.wait()

            for t in range(2 * (N_DEV - 1)):
                s_slot = t % 2
                r_slot = (t + 1) % 2
                if step > 0:
                    pl.semaphore_wait(credit_sem, 1)
                rdma = pltpu.make_async_remote_copy(
                    src_ref=comm.at[s_slot],
                    dst_ref=comm.at[r_slot],
                    send_sem=send_sems.at[s_slot],
                    recv_sem=recv_sems.at[r_slot],
                    device_id=(right,),
                    device_id_type=pl.DeviceIdType.MESH,
                )
                rdma.start()
                rdma.wait()

                rc = jnp.mod(my - t - 1, N_DEV)
                goff = (g * N_DEV + rc) * ROWS
                if t < N_DEV - 1:
                    ld = pltpu.make_async_copy(
                        p_ref.at[pl.ds(goff, ROWS)], local, copy_sem)
                    ld.start()
                    ld.wait()
                    acc = comm[r_slot] + local[...]
                    if t == N_DEV - 2:
                        acc = _silu(acc)
                    comm[r_slot] = acc
                if t >= N_DEV - 2:
                    st = pltpu.make_async_copy(
                        comm.at[r_slot], o_ref.at[pl.ds(goff, ROWS)],
                        store_sem)
                    st.start()
                    st.wait()
                pl.semaphore_signal(credit_sem, inc=1, device_id=(left,),
                                    device_id_type=pl.DeviceIdType.MESH)
                step += 1

    return pl.pallas_call(
        body,
        out_shape=jax.ShapeDtypeStruct((M, N), jnp.float32),
        in_specs=[pl.BlockSpec(memory_space=pltpu.ANY)],
        out_specs=pl.BlockSpec(memory_space=pltpu.ANY),
        scratch_shapes=[
            pltpu.VMEM((2, ROWS, N), jnp.float32),
            pltpu.VMEM((ROWS, N), jnp.float32),
            pltpu.SemaphoreType.DMA((2,)),
            pltpu.SemaphoreType.DMA((2,)),
            pltpu.SemaphoreType.DMA,
            pltpu.SemaphoreType.DMA,
            pltpu.SemaphoreType.REGULAR,
        ],
        compiler_params=pltpu.CompilerParams(collective_id=0),
    )(p)


# baseline (device time: 2505679 ns/iter reference)
import jax
import jax.numpy as jnp
from jax import lax
from jax.experimental import pallas as pl
from jax.experimental.pallas import tpu as pltpu

N_DEV = 4
M, N = 4096, 8192
ROWS = 256
CHUNKS = M // ROWS
ROUNDS = CHUNKS // N_DEV


def _silu(y):
    return y * (1.0 / (1.0 + jnp.exp(-y)))


def kernel(x, w_mat):
    p = jnp.dot(x, w_mat, preferred_element_type=jnp.float32)

    def body(p_ref, o_ref, comm, local, send_sems, recv_sems, copy_sem,
             store_sem, credit_sem):
        my = lax.axis_index("i")
        left = jnp.mod(my - 1, N_DEV)
        right = jnp.mod(my + 1, N_DEV)

        barrier = pltpu.get_barrier_semaphore()
        for nbr in (left, right):
            pl.semaphore_signal(barrier, inc=1, device_id=(nbr,),
                                device_id_type=pl.DeviceIdType.MESH)
        pl.semaphore_wait(barrier, 2)

        step = 0
        for g in range(ROUNDS):
            ld = pltpu.make_async_copy(
                p_ref.at[pl.ds((g * N_DEV + my) * ROWS, ROWS)],
                comm.at[0], copy_sem)
            ld.start()
            ld.wait()

            for t in range(2 * (N_DEV - 1)):
                s_slot = t % 2
                r_slot = (t + 1) % 2
                if step > 0:
                    pl.semaphore_wait(credit_sem, 1)
                rdma = pltpu.make_async_remote_copy(
                    src_ref=comm.at[s_slot],
                    dst_ref=comm.at[r_slot],
                    send_sem=send_sems.at[s_slot],
                    recv_sem=recv_sems.at[r_slot],
                    device_id=(right,),
                    device_id_type=pl.DeviceIdType.MESH,
                )
                rdma.start()
                rdma.wait()

                rc = jnp.mod(my - t - 1, N_DEV)
                goff = (g * N_DEV + rc) * ROWS
                if t < N_DEV - 1:
                    ld = pltpu.make_async_copy(
                        p_ref.at[pl.ds(goff, ROWS)], local, copy_sem)
                    ld.start()
                    ld.wait()
                    acc = comm[r_slot] + local[...]
                    if t == N_DEV - 2:
                        acc = _silu(acc)
                    comm[r_slot] = acc
                if t >= N_DEV - 2:
                    st = pltpu.make_async_copy(
                        comm.at[r_slot], o_ref.at[pl.ds(goff, ROWS)],
                        store_sem)
                    st.start()
                    st.wait()
                step += 1
                if step < ROUNDS * 2 * (N_DEV - 1):
                    pl.semaphore_signal(credit_sem, inc=1, device_id=(left,),
                                        device_id_type=pl.DeviceIdType.MESH)

    return pl.pallas_call(
        body,
        out_shape=jax.ShapeDtypeStruct((M, N), jnp.float32),
        in_specs=[pl.BlockSpec(memory_space=pl.ANY)],
        out_specs=pl.BlockSpec(memory_space=pl.ANY),
        scratch_shapes=[
            pltpu.VMEM((2, ROWS, N), jnp.float32),
            pltpu.VMEM((ROWS, N), jnp.float32),
            pltpu.SemaphoreType.DMA((2,)),
            pltpu.SemaphoreType.DMA((2,)),
            pltpu.SemaphoreType.DMA,
            pltpu.SemaphoreType.DMA,
            pltpu.SemaphoreType.REGULAR,
        ],
        compiler_params=pltpu.CompilerParams(collective_id=0),
    )(p)


# device time: 1384929 ns/iter; 1.8092x vs baseline; 1.8092x over previous
import jax
import jax.numpy as jnp
from jax import lax
from jax.experimental import pallas as pl
from jax.experimental.pallas import tpu as pltpu

N_DEV = 4
M, N = 4096, 8192
HALF = M // 2
ROWS = 128
ROUNDS = HALF // (N_DEV * ROWS)
STEPS = 2 * (N_DEV - 1)


def _silu(y):
    return y * (1.0 / (1.0 + jnp.exp(-y)))


def kernel(x, w_mat):
    p = jnp.dot(x, w_mat, preferred_element_type=jnp.float32)

    def body(p_ref, o_ref, comm_r, comm_l, local_r, local_l,
             send_sems_r, recv_sems_r, send_sems_l, recv_sems_l,
             copy_sem_r, copy_sem_l, store_sem_r, store_sem_l,
             credit_r, credit_l):
        my = lax.axis_index("i")
        left = jnp.mod(my - 1, N_DEV)
        right = jnp.mod(my + 1, N_DEV)

        barrier = pltpu.get_barrier_semaphore()
        for nbr in (left, right):
            pl.semaphore_signal(barrier, inc=1, device_id=(nbr,),
                                device_id_type=pl.DeviceIdType.MESH)
        pl.semaphore_wait(barrier, 2)

        step = 0
        last_step = ROUNDS * STEPS
        for g in range(ROUNDS):
            base = g * N_DEV * ROWS
            ld = pltpu.make_async_copy(
                p_ref.at[pl.ds(base + my * ROWS, ROWS)],
                comm_r.at[0], copy_sem_r)
            ld.start()
            ld2 = pltpu.make_async_copy(
                p_ref.at[pl.ds(HALF + base + my * ROWS, ROWS)],
                comm_l.at[0], copy_sem_l)
            ld2.start()
            ld.wait()
            ld2.wait()

            for t in range(STEPS):
                s_slot = t % 2
                r_slot = (t + 1) % 2
                if step > 0:
                    pl.semaphore_wait(credit_r, 1)
                    pl.semaphore_wait(credit_l, 1)
                rdma_r = pltpu.make_async_remote_copy(
                    src_ref=comm_r.at[s_slot],
                    dst_ref=comm_r.at[r_slot],
                    send_sem=send_sems_r.at[s_slot],
                    recv_sem=recv_sems_r.at[r_slot],
                    device_id=(right,),
                    device_id_type=pl.DeviceIdType.MESH,
                )
                rdma_l = pltpu.make_async_remote_copy(
                    src_ref=comm_l.at[s_slot],
                    dst_ref=comm_l.at[r_slot],
                    send_sem=send_sems_l.at[s_slot],
                    recv_sem=recv_sems_l.at[r_slot],
                    device_id=(left,),
                    device_id_type=pl.DeviceIdType.MESH,
                )
                rdma_r.start()
                rdma_l.start()

                rc_r = jnp.mod(my - t - 1, N_DEV)
                rc_l = jnp.mod(my + t + 1, N_DEV)
                goff_r = base + rc_r * ROWS
                goff_l = HALF + base + rc_l * ROWS

                if t < N_DEV - 1:
                    ld_r = pltpu.make_async_copy(
                        p_ref.at[pl.ds(goff_r, ROWS)], local_r, copy_sem_r)
                    ld_l = pltpu.make_async_copy(
                        p_ref.at[pl.ds(goff_l, ROWS)], local_l, copy_sem_l)
                    ld_r.start()
                    ld_l.start()

                rdma_r.wait()
                rdma_l.wait()

                if t < N_DEV - 1:
                    ld_r.wait()
                    ld_l.wait()
                    acc_r = comm_r[r_slot] + local_r[...]
                    acc_l = comm_l[r_slot] + local_l[...]
                    if t == N_DEV - 2:
                        acc_r = _silu(acc_r)
                        acc_l = _silu(acc_l)
                    comm_r[r_slot] = acc_r
                    comm_l[r_slot] = acc_l
                if t >= N_DEV - 2:
                    st_r = pltpu.make_async_copy(
                        comm_r.at[r_slot], o_ref.at[pl.ds(goff_r, ROWS)],
                        store_sem_r)
                    st_l = pltpu.make_async_copy(
                        comm_l.at[r_slot], o_ref.at[pl.ds(goff_l, ROWS)],
                        store_sem_l)
                    st_r.start()
                    st_l.start()
                    st_r.wait()
                    st_l.wait()

                step += 1
                if step < last_step:
                    pl.semaphore_signal(credit_r, inc=1, device_id=(left,),
                                        device_id_type=pl.DeviceIdType.MESH)
                    pl.semaphore_signal(credit_l, inc=1, device_id=(right,),
                                        device_id_type=pl.DeviceIdType.MESH)

    return pl.pallas_call(
        body,
        out_shape=jax.ShapeDtypeStruct((M, N), jnp.float32),
        in_specs=[pl.BlockSpec(memory_space=pl.ANY)],
        out_specs=pl.BlockSpec(memory_space=pl.ANY),
        scratch_shapes=[
            pltpu.VMEM((2, ROWS, N), jnp.float32),
            pltpu.VMEM((2, ROWS, N), jnp.float32),
            pltpu.VMEM((ROWS, N), jnp.float32),
            pltpu.VMEM((ROWS, N), jnp.float32),
            pltpu.SemaphoreType.DMA((2,)),
            pltpu.SemaphoreType.DMA((2,)),
            pltpu.SemaphoreType.DMA((2,)),
            pltpu.SemaphoreType.DMA((2,)),
            pltpu.SemaphoreType.DMA,
            pltpu.SemaphoreType.DMA,
            pltpu.SemaphoreType.DMA,
            pltpu.SemaphoreType.DMA,
            pltpu.SemaphoreType.REGULAR,
            pltpu.SemaphoreType.REGULAR,
        ],
        compiler_params=pltpu.CompilerParams(collective_id=0),
    )(p)


# device time: 1255949 ns/iter; 1.9950x vs baseline; 1.1027x over previous
import jax
import jax.numpy as jnp
from jax import lax
from jax.experimental import pallas as pl
from jax.experimental.pallas import tpu as pltpu

N_DEV = 4
M, N = 4096, 8192
N_PIPES = 4
PIPE_ROWS = M // N_PIPES
ROWS = 64
ROUNDS = PIPE_ROWS // (N_DEV * ROWS)
STEPS = 2 * (N_DEV - 1)
TOTAL = ROUNDS * STEPS
PIPE_DIR = (1, 1, -1, -1)
PIPE_BASE = (0, PIPE_ROWS, 2 * PIPE_ROWS, 3 * PIPE_ROWS)
PIPE_ORDER = (0, 2, 1, 3)


def _silu(y):
    return y * (1.0 / (1.0 + jnp.exp(-y)))


def kernel(x, w_mat):
    p = jnp.dot(x, w_mat, preferred_element_type=jnp.float32)

    def body(p_ref, o_ref, *scr):
        comms = scr[0:4]
        locs = scr[4:8]
        ssems = scr[8:12]
        rsems = scr[12:16]
        csems = scr[16:20]
        stsems = scr[20:24]
        creds = scr[24:28]

        my = lax.axis_index("i")
        left = jnp.mod(my - 1, N_DEV)
        right = jnp.mod(my + 1, N_DEV)

        barrier = pltpu.get_barrier_semaphore()
        for nbr in (left, right):
            pl.semaphore_signal(barrier, inc=1, device_id=(nbr,),
                                device_id_type=pl.DeviceIdType.MESH)
        pl.semaphore_wait(barrier, 2)

        def tgt(pi):
            return right if PIPE_DIR[pi] > 0 else left

        def src(pi):
            return left if PIPE_DIR[pi] > 0 else right

        def rchunk(pi, u):
            t = u % STEPS
            return jnp.mod(my - PIPE_DIR[pi] * (t + 1), N_DEV)

        def goff(pi, u):
            return PIPE_BASE[pi] + ((u // STEPS) * N_DEV + rchunk(pi, u)) * ROWS

        def mk_rdma(pi, u):
            s_slot, r_slot = u % 2, (u + 1) % 2
            return pltpu.make_async_remote_copy(
                src_ref=comms[pi].at[s_slot],
                dst_ref=comms[pi].at[r_slot],
                send_sem=ssems[pi].at[s_slot],
                recv_sem=rsems[pi].at[r_slot],
                device_id=(tgt(pi),),
                device_id_type=pl.DeviceIdType.MESH,
            )

        def start_load(pi, dst, off):
            c = pltpu.make_async_copy(p_ref.at[pl.ds(off, ROWS)], dst,
                                      csems[pi])
            c.start()
            return c

        init = [start_load(pi, comms[pi].at[0], PIPE_BASE[pi] + my * ROWS)
                for pi in range(N_PIPES)]
        rdmas = [None] * N_PIPES
        loads = [None] * N_PIPES
        stores = [None] * N_PIPES
        for pi in range(N_PIPES):
            init[pi].wait()
            rdmas[pi] = mk_rdma(pi, 0)
            rdmas[pi].start()
            loads[pi] = start_load(pi, locs[pi], goff(pi, 0))

        for u in range(TOTAL):
            t = u % STEPS
            g = u // STEPS
            r_slot = (u + 1) % 2
            for pi in PIPE_ORDER:
                rdmas[pi].wait()
                if stores[pi] is not None:
                    stores[pi].wait()
                    stores[pi] = None
                if u < TOTAL - 1:
                    pl.semaphore_signal(creds[pi], inc=1,
                                        device_id=(src(pi),),
                                        device_id_type=pl.DeviceIdType.MESH)
                off = goff(pi, u)
                if t < N_DEV - 1:
                    loads[pi].wait()
                    loads[pi] = None
                    acc = comms[pi][r_slot] + locs[pi][...]
                    if t == N_DEV - 2:
                        acc = _silu(acc)
                    comms[pi][r_slot] = acc
                if t >= N_DEV - 2:
                    stc = pltpu.make_async_copy(
                        comms[pi].at[r_slot],
                        o_ref.at[pl.ds(off, ROWS)], stsems[pi])
                    stc.start()
                    stores[pi] = stc
                if u + 1 < TOTAL:
                    t2 = (u + 1) % STEPS
                    if t2 == 0:
                        if stores[pi] is not None:
                            stores[pi].wait()
                            stores[pi] = None
                        c0 = start_load(
                            pi, comms[pi].at[0],
                            PIPE_BASE[pi] + ((g + 1) * N_DEV + my) * ROWS)
                        c0.wait()
                    pl.semaphore_wait(creds[pi], 1)
                    rdmas[pi] = mk_rdma(pi, u + 1)
                    rdmas[pi].start()
                    if t2 < N_DEV - 1:
                        loads[pi] = start_load(pi, locs[pi], goff(pi, u + 1))

        for pi in PIPE_ORDER:
            if stores[pi] is not None:
                stores[pi].wait()

    return pl.pallas_call(
        body,
        out_shape=jax.ShapeDtypeStruct((M, N), jnp.float32),
        in_specs=[pl.BlockSpec(memory_space=pl.ANY)],
        out_specs=pl.BlockSpec(memory_space=pl.ANY),
        scratch_shapes=(
            [pltpu.VMEM((2, ROWS, N), jnp.float32)] * N_PIPES
            + [pltpu.VMEM((ROWS, N), jnp.float32)] * N_PIPES
            + [pltpu.SemaphoreType.DMA((2,))] * N_PIPES
            + [pltpu.SemaphoreType.DMA((2,))] * N_PIPES
            + [pltpu.SemaphoreType.DMA] * N_PIPES
            + [pltpu.SemaphoreType.DMA] * N_PIPES
            + [pltpu.SemaphoreType.REGULAR] * N_PIPES
        ),
        compiler_params=pltpu.CompilerParams(collective_id=0),
    )(p)
